# baseline (speedup 1.0000x reference)
import jax
import jax.numpy as jnp
from jax.experimental import pallas as pl
from jax.experimental.pallas import tpu as pltpu

_N = 1024
_H = 8


def _auction_body(eps_ref, iters_ref, x1_ref, x2t_ref, dist_ref, ass_ref,
                  c_ref, unass_ref, maxinc_ref, winner_ref):
    n = _N
    x1 = x1_ref[0]
    x2t = x2t_ref[0]
    eps = eps_ref[0]
    iters = iters_ref[0]

    d0 = x1[:, 0:1] - x2t[0:1, :]
    d1 = x1[:, 1:2] - x2t[1:2, :]
    d2 = x1[:, 2:3] - x2t[2:3, :]
    c_ref[...] = (d0 * d0 + d1 * d1) + d2 * d2

    col = jax.lax.broadcasted_iota(jnp.int32, (1, n), 1)
    row = jax.lax.broadcasted_iota(jnp.int32, (n, 1), 0)
    row_h = jax.lax.broadcasted_iota(jnp.int32, (_H, 1), 0)
    neg_inf = jnp.float32(-jnp.inf)

    unass_ref[...] = jnp.ones((n, 1), jnp.int32)

    def round_body(_, carry):
        price, ass_inv = carry
        pneg = -price
        maxinc_ref[...] = jnp.full((1, n), neg_inf)
        winner_ref[...] = jnp.full((1, n), n, jnp.int32)

        def block_body(ib, _c):
            ib8 = pl.multiple_of(ib * _H, _H)
            blk = pl.ds(ib8, _H)
            ub = unass_ref[blk, :] > 0

            @pl.when(jnp.any(ub))
            def _():
                vb = pneg - c_ref[blk, :]
                best = jnp.max(vb, axis=1, keepdims=True)
                iseq = vb == best
                cnt = jnp.sum(iseq.astype(jnp.int32), axis=1, keepdims=True)
                bidx = jnp.min(jnp.where(iseq, col, n), axis=1, keepdims=True)
                m2 = jnp.max(jnp.where(iseq, neg_inf, vb), axis=1, keepdims=True)
                second = jnp.where(cnt > 1, best, m2)
                binc = best - second + eps
                bm = (bidx == col) & ub
                bb = jnp.where(bm, binc, neg_inf)
                bmax = jnp.max(bb, axis=0, keepdims=True)
                rows = row_h + ib8
                bwin = jnp.min(jnp.where(bb == bmax, rows, n), axis=0,
                               keepdims=True)
                cur = maxinc_ref[...]
                curw = winner_ref[...]
                better = bmax > cur
                eqv = bmax == cur
                winner_ref[...] = jnp.where(
                    better, bwin,
                    jnp.where(eqv, jnp.minimum(curw, bwin), curw))
                maxinc_ref[...] = jnp.maximum(cur, bmax)

            return _c

        jax.lax.fori_loop(0, n // _H, block_body, 0, unroll=False)

        maxinc = maxinc_ref[...]
        has_bid = maxinc > neg_inf
        price2 = jnp.where(has_bid, price + maxinc, price)
        ass_inv2 = jnp.where(has_bid, winner_ref[...], ass_inv)
        owned = jnp.any(ass_inv2 == row, axis=1, keepdims=True)
        unass_ref[...] = 1 - owned.astype(jnp.int32)
        return price2, ass_inv2

    price0 = jnp.zeros((1, n), jnp.float32)
    ass_inv0 = jnp.full((1, n), -1, jnp.int32)
    _, ass_inv = jax.lax.fori_loop(0, iters, round_body, (price0, ass_inv0))

    ass = jnp.max(jnp.where(ass_inv == row, col, jnp.int32(-1)),
                  axis=1, keepdims=True)
    dist = jnp.max(jnp.where(ass == col, c_ref[...], 0.0), axis=1,
                   keepdims=True)
    dist_ref[0] = dist
    ass_ref[0] = ass


def kernel(input1, input2, eps, iters):
    b, n, _ = input1.shape
    x2t = jnp.transpose(input2, (0, 2, 1))
    eps_a = jnp.asarray(eps, jnp.float32).reshape(1)
    it_a = jnp.asarray(iters, jnp.int32).reshape(1)
    dist3, ass3 = pl.pallas_call(
        _auction_body,
        grid=(b,),
        in_specs=[
            pl.BlockSpec(memory_space=pltpu.SMEM),
            pl.BlockSpec(memory_space=pltpu.SMEM),
            pl.BlockSpec((1, n, 3), lambda i: (i, 0, 0)),
            pl.BlockSpec((1, 3, n), lambda i: (i, 0, 0)),
        ],
        out_specs=[
            pl.BlockSpec((1, n, 1), lambda i: (i, 0, 0)),
            pl.BlockSpec((1, n, 1), lambda i: (i, 0, 0)),
        ],
        out_shape=[
            jax.ShapeDtypeStruct((b, n, 1), jnp.float32),
            jax.ShapeDtypeStruct((b, n, 1), jnp.int32),
        ],
        scratch_shapes=[
            pltpu.VMEM((n, n), jnp.float32),
            pltpu.VMEM((n, 1), jnp.int32),
            pltpu.VMEM((1, n), jnp.float32),
            pltpu.VMEM((1, n), jnp.int32),
        ],
    )(eps_a, it_a, input1, x2t)
    return dist3[..., 0], ass3[..., 0]


# speedup vs baseline: 7.5540x; 7.5540x over previous
import jax
import jax.numpy as jnp
from jax.experimental import pallas as pl
from jax.experimental.pallas import tpu as pltpu

_N = 1024


def _auction_body(eps_ref, iters_ref, x1_ref, x2t_ref, dist_ref, ass_ref, c_ref):
    n = _N
    x1 = x1_ref[0]
    x2t = x2t_ref[0]
    eps = eps_ref[0]
    iters = iters_ref[0]

    d0 = x1[:, 0:1] - x2t[0:1, :]
    d1 = x1[:, 1:2] - x2t[1:2, :]
    d2 = x1[:, 2:3] - x2t[2:3, :]
    c_ref[...] = (d0 * d0 + d1 * d1) + d2 * d2

    col = jax.lax.broadcasted_iota(jnp.int32, (1, n), 1)
    row = jax.lax.broadcasted_iota(jnp.int32, (n, 1), 0)
    neg_inf = jnp.float32(-jnp.inf)

    def round_body(_, carry):
        price, ass_inv, unass_i = carry
        unass = unass_i > 0
        pneg = -price
        vb = pneg - c_ref[...]
        best = jnp.max(vb, axis=1, keepdims=True)
        iseq = vb == best
        cnt = jnp.sum(iseq.astype(jnp.int32), axis=1, keepdims=True)
        bidx = jnp.min(jnp.where(iseq, col, n), axis=1, keepdims=True)
        m2 = jnp.max(jnp.where(iseq, neg_inf, vb), axis=1, keepdims=True)
        second = jnp.where(cnt > 1, best, m2)
        binc = best - second + eps
        bm = (bidx == col) & unass
        bb = jnp.where(bm, binc, neg_inf)
        maxinc = jnp.max(bb, axis=0, keepdims=True)
        winner = jnp.min(jnp.where(bb == maxinc, row, n), axis=0,
                         keepdims=True)
        has_bid = maxinc > neg_inf
        price2 = jnp.where(has_bid, price + maxinc, price)
        ass_inv2 = jnp.where(has_bid, winner, ass_inv)
        owned = jnp.any(ass_inv2 == row, axis=1, keepdims=True)
        return price2, ass_inv2, 1 - owned.astype(jnp.int32)

    price0 = jnp.zeros((1, n), jnp.float32)
    ass_inv0 = jnp.full((1, n), -1, jnp.int32)
    unass0 = jnp.ones((n, 1), jnp.int32)
    _, ass_inv, _ = jax.lax.fori_loop(
        0, iters, round_body, (price0, ass_inv0, unass0))

    ass = jnp.max(jnp.where(ass_inv == row, col, jnp.int32(-1)),
                  axis=1, keepdims=True)
    dist = jnp.max(jnp.where(ass == col, c_ref[...], 0.0), axis=1,
                   keepdims=True)
    dist_ref[0] = dist
    ass_ref[0] = ass


def kernel(input1, input2, eps, iters):
    b, n, _ = input1.shape
    x2t = jnp.transpose(input2, (0, 2, 1))
    eps_a = jnp.asarray(eps, jnp.float32).reshape(1)
    it_a = jnp.asarray(iters, jnp.int32).reshape(1)
    dist3, ass3 = pl.pallas_call(
        _auction_body,
        grid=(b,),
        in_specs=[
            pl.BlockSpec(memory_space=pltpu.SMEM),
            pl.BlockSpec(memory_space=pltpu.SMEM),
            pl.BlockSpec((1, n, 3), lambda i: (i, 0, 0)),
            pl.BlockSpec((1, 3, n), lambda i: (i, 0, 0)),
        ],
        out_specs=[
            pl.BlockSpec((1, n, 1), lambda i: (i, 0, 0)),
            pl.BlockSpec((1, n, 1), lambda i: (i, 0, 0)),
        ],
        out_shape=[
            jax.ShapeDtypeStruct((b, n, 1), jnp.float32),
            jax.ShapeDtypeStruct((b, n, 1), jnp.int32),
        ],
        scratch_shapes=[pltpu.VMEM((n, n), jnp.float32)],
    )(eps_a, it_a, input1, x2t)
    return dist3[..., 0], ass3[..., 0]
